# Initial kernel scaffold; baseline (speedup 1.0000x reference)
import jax
import jax.numpy as jnp
from jax.experimental import pallas as pl


def kernel(weight, load, adj_indices, adj_values, wire_indices, wire_values):
    raise NotImplementedError("write your pallas kernel here")


# speedup vs baseline: 128.4466x; 128.4466x over previous
import functools

import jax
import jax.numpy as jnp
from jax import lax
from jax.experimental import pallas as pl
from jax.experimental.pallas import tpu as pltpu
from jax.experimental.pallas import tpu_sc as plsc

L = 64
MAXNODE = 512
MAXFANOUT = 8
N0 = 2 * L * MAXNODE * MAXFANOUT
NNZ = 2097152

NC = 2
NS = 16
NW = NC * NS
EPT = NNZ // NW
CH = 2048
NCH = EPT // CH
ACC_T = N0 // NS


def _make_sc_scatter(table_size: int):
    tslice = table_size // NS

    def body(gidx_h, sidx_h, val_h, tbl_h, out_h,
             tbl_s, acc_s, gq, sq, vq, wq, pq, zb, sem):
        c = lax.axis_index("c")
        s = lax.axis_index("s")
        wid = c * NS + s

        def zstep(i, carry):
            zb[pl.ds(i * 16, 16)] = jnp.zeros((16,), jnp.float32)
            return carry
        lax.fori_loop(0, CH // 16, zstep, 0)
        pltpu.sync_copy(tbl_h.at[pl.ds(s * tslice, tslice)],
                        tbl_s.at[pl.ds(s * tslice, tslice)])
        for t in range(ACC_T // CH):
            pltpu.sync_copy(zb, acc_s.at[pl.ds(s * ACC_T + t * CH, CH)])
        plsc.subcore_barrier()

        def chunk(j, carry):
            base = wid * EPT + j * CH
            pltpu.sync_copy(gidx_h.at[pl.ds(base, CH)], gq)
            pltpu.sync_copy(sidx_h.at[pl.ds(base, CH)], sq)
            pltpu.sync_copy(val_h.at[pl.ds(base, CH)], vq)
            pltpu.async_copy(tbl_s.at[gq], wq, sem).wait()

            def mstep(i, c2):
                sl = pl.ds(i * 16, 16)
                pq[sl] = wq[sl] * vq[sl]
                return c2
            lax.fori_loop(0, CH // 16, mstep, 0)
            pltpu.sync_copy(pq, acc_s.at[sq], add=True)
            return carry
        lax.fori_loop(0, NCH, chunk, 0)
        plsc.subcore_barrier()

        pltpu.sync_copy(acc_s.at[pl.ds(s * ACC_T, ACC_T)],
                        out_h.at[pl.ds(c * N0 + s * ACC_T, ACC_T)])

    mesh = plsc.VectorSubcoreMesh(core_axis_name="c", subcore_axis_name="s")
    return pl.kernel(
        body,
        out_type=jax.ShapeDtypeStruct((NC * N0,), jnp.float32),
        mesh=mesh,
        scratch_types=[
            pltpu.VMEM_SHARED((table_size,), jnp.float32),
            pltpu.VMEM_SHARED((N0,), jnp.float32),
            pltpu.VMEM((CH,), jnp.int32),
            pltpu.VMEM((CH,), jnp.int32),
            pltpu.VMEM((CH,), jnp.float32),
            pltpu.VMEM((CH,), jnp.float32),
            pltpu.VMEM((CH,), jnp.float32),
            pltpu.VMEM((CH,), jnp.float32),
            pltpu.SemaphoreType.DMA,
        ],
    )


def _group_sum_mat():
    l = lax.broadcasted_iota(jnp.int32, (128, 16), 0)
    g = lax.broadcasted_iota(jnp.int32, (128, 16), 1)
    return (l // 8 == g).astype(jnp.float32)


def _dense1_body(p0, p1, ld, o_ref, wl_ref):
    o = p0[...] + p1[...]
    o_ref[...] = o
    wl_ref[...] = jnp.dot(ld[...] * o, _group_sum_mat(),
                          preferred_element_type=jnp.float32)


def _dense2_body(q0, q1, o, wl, out_ref):
    lw = (q0[...] + q1[...]) * o[...]
    out_ref[...] = wl[...] + jnp.dot(lw, _group_sum_mat(),
                                     preferred_element_type=jnp.float32)


_R = N0 // 128
_BR = 512
_G = 8


def _dense1(p0, p1, ld):
    big = pl.BlockSpec((_BR, 128), lambda i: (i, 0))
    small = pl.BlockSpec((_BR, 16), lambda i: (i, 0))
    return pl.pallas_call(
        _dense1_body,
        grid=(_G,),
        in_specs=[big, big, big],
        out_specs=[big, small],
        out_shape=[jax.ShapeDtypeStruct((_R, 128), jnp.float32),
                   jax.ShapeDtypeStruct((_R, 16), jnp.float32)],
    )(p0, p1, ld)


def _dense2(q0, q1, o, wl):
    big = pl.BlockSpec((_BR, 128), lambda i: (i, 0))
    small = pl.BlockSpec((_BR, 16), lambda i: (i, 0))
    return pl.pallas_call(
        _dense2_body,
        grid=(_G,),
        in_specs=[big, big, big, small],
        out_specs=small,
        out_shape=jax.ShapeDtypeStruct((_R, 16), jnp.float32),
    )(q0, q1, o, wl)


@jax.jit
def kernel(weight, load, adj_indices, adj_values, wire_indices, wire_values):
    gidx = ((adj_indices[1] * L + adj_indices[2]) * MAXNODE
            + adj_indices[3]) * MAXFANOUT + adj_indices[4]
    wgidx = (wire_indices[1] * L + wire_indices[2]) * MAXNODE + wire_indices[3]

    sc1 = _make_sc_scatter(N0)
    p = sc1(gidx, adj_indices[0], adj_values, weight.reshape(-1))
    p = p.reshape(NC, _R, 128)
    o, wl = _dense1(p[0], p[1], load.reshape(_R, 128))

    sc2 = _make_sc_scatter(N0 // MAXFANOUT)
    q = sc2(wgidx, wire_indices[0], wire_values, wl.reshape(-1))
    q = q.reshape(NC, _R, 128)
    out = _dense2(q[0], q[1], o, wl)
    return out.reshape(2, L, MAXNODE)


# speedup vs baseline: 150.8628x; 1.1745x over previous
import jax
import jax.numpy as jnp
from jax import lax
from jax.experimental import pallas as pl
from jax.experimental.pallas import tpu as pltpu
from jax.experimental.pallas import tpu_sc as plsc

L = 64
MAXNODE = 512
MAXFANOUT = 8
N0 = 2 * L * MAXNODE * MAXFANOUT
NNZ = 2097152

NC = 2
NS = 16
NW = NC * NS
EPT = NNZ // NW
CH = 4096
NCH = EPT // CH
ACC_T = N0 // NS


def _make_sc_scatter(table_size: int, table_in_vmem: bool):
    tslice = table_size // NS

    def body(gidx_h, sidx_h, val_h, tbl_h, out_h,
             acc_s, tbl_r, gq, sq, vq, pq, zb, sem, *maybe_wq):
        c = lax.axis_index("c")
        s = lax.axis_index("s")
        wid = c * NS + s

        def zstep(i, carry):
            zb[pl.ds(i * 16, 16)] = jnp.zeros((16,), jnp.float32)
            return carry
        lax.fori_loop(0, CH // 16, zstep, 0)
        if table_in_vmem:
            pltpu.sync_copy(tbl_h, tbl_r)
        else:
            pltpu.sync_copy(tbl_h.at[pl.ds(s * tslice, tslice)],
                            tbl_r.at[pl.ds(s * tslice, tslice)])
        for t in range(ACC_T // CH):
            pltpu.sync_copy(zb, acc_s.at[pl.ds(s * ACC_T + t * CH, CH)])
        plsc.subcore_barrier()

        def chunk(j, carry):
            base = wid * EPT + j * CH
            pltpu.sync_copy(gidx_h.at[pl.ds(base, CH)], gq)
            pltpu.sync_copy(sidx_h.at[pl.ds(base, CH)], sq)
            pltpu.sync_copy(val_h.at[pl.ds(base, CH)], vq)
            if table_in_vmem:
                def mstep(i, c2):
                    sl = pl.ds(i * 16, 16)
                    w = plsc.load_gather(tbl_r, [gq[sl]])
                    pq[sl] = w * vq[sl]
                    return c2
            else:
                wq = maybe_wq[0]
                pltpu.async_copy(tbl_r.at[gq], wq, sem).wait()

                def mstep(i, c2):
                    sl = pl.ds(i * 16, 16)
                    pq[sl] = wq[sl] * vq[sl]
                    return c2
            lax.fori_loop(0, CH // 16, mstep, 0)
            pltpu.sync_copy(pq, acc_s.at[sq], add=True)
            return carry
        lax.fori_loop(0, NCH, chunk, 0)
        plsc.subcore_barrier()

        pltpu.sync_copy(acc_s.at[pl.ds(s * ACC_T, ACC_T)],
                        out_h.at[pl.ds(c * N0 + s * ACC_T, ACC_T)])

    mesh = plsc.VectorSubcoreMesh(core_axis_name="c", subcore_axis_name="s")
    tbl_scratch = (pltpu.VMEM((table_size,), jnp.float32) if table_in_vmem
                   else pltpu.VMEM_SHARED((table_size,), jnp.float32))
    scratch = [
        pltpu.VMEM_SHARED((N0,), jnp.float32),
        tbl_scratch,
        pltpu.VMEM((CH,), jnp.int32),
        pltpu.VMEM((CH,), jnp.int32),
        pltpu.VMEM((CH,), jnp.float32),
        pltpu.VMEM((CH,), jnp.float32),
        pltpu.VMEM((CH,), jnp.float32),
        pltpu.SemaphoreType.DMA,
    ]
    if not table_in_vmem:
        scratch.append(pltpu.VMEM((CH,), jnp.float32))
    return pl.kernel(
        body,
        out_type=jax.ShapeDtypeStruct((NC * N0,), jnp.float32),
        mesh=mesh,
        compiler_params=pltpu.CompilerParams(needs_layout_passes=False),
        scratch_types=scratch,
    )


def _group_sum_mat():
    l = lax.broadcasted_iota(jnp.int32, (128, 16), 0)
    g = lax.broadcasted_iota(jnp.int32, (128, 16), 1)
    return (l // 8 == g).astype(jnp.float32)


def _dense1_body(p0, p1, ld, o_ref, wl_ref):
    o = p0[...] + p1[...]
    o_ref[...] = o
    wl_ref[...] = jnp.dot(ld[...] * o, _group_sum_mat(),
                          preferred_element_type=jnp.float32)


def _dense2_body(q0, q1, o, wl, out_ref):
    lw = (q0[...] + q1[...]) * o[...]
    out_ref[...] = wl[...] + jnp.dot(lw, _group_sum_mat(),
                                     preferred_element_type=jnp.float32)


_R = N0 // 128
_BR = 512
_G = 8


def _dense1(p0, p1, ld):
    big = pl.BlockSpec((_BR, 128), lambda i: (i, 0))
    small = pl.BlockSpec((_BR, 16), lambda i: (i, 0))
    return pl.pallas_call(
        _dense1_body,
        grid=(_G,),
        in_specs=[big, big, big],
        out_specs=[big, small],
        out_shape=[jax.ShapeDtypeStruct((_R, 128), jnp.float32),
                   jax.ShapeDtypeStruct((_R, 16), jnp.float32)],
    )(p0, p1, ld)


def _dense2(q0, q1, o, wl):
    big = pl.BlockSpec((_BR, 128), lambda i: (i, 0))
    small = pl.BlockSpec((_BR, 16), lambda i: (i, 0))
    return pl.pallas_call(
        _dense2_body,
        grid=(_G,),
        in_specs=[big, big, big, small],
        out_specs=small,
        out_shape=jax.ShapeDtypeStruct((_R, 16), jnp.float32),
    )(q0, q1, o, wl)


@jax.jit
def kernel(weight, load, adj_indices, adj_values, wire_indices, wire_values):
    gidx = ((adj_indices[1] * L + adj_indices[2]) * MAXNODE
            + adj_indices[3]) * MAXFANOUT + adj_indices[4]
    wgidx = (wire_indices[1] * L + wire_indices[2]) * MAXNODE + wire_indices[3]

    sc1 = _make_sc_scatter(N0, table_in_vmem=False)
    p = sc1(gidx, adj_indices[0], adj_values, weight.reshape(-1))
    p = p.reshape(NC, _R, 128)
    o, wl = _dense1(p[0], p[1], load.reshape(_R, 128))

    sc2 = _make_sc_scatter(N0 // MAXFANOUT, table_in_vmem=True)
    q = sc2(wgidx, wire_indices[0], wire_values, wl.reshape(-1))
    q = q.reshape(NC, _R, 128)
    out = _dense2(q[0], q[1], o, wl)
    return out.reshape(2, L, MAXNODE)
